# Initial kernel scaffold; baseline (speedup 1.0000x reference)
import jax
import jax.numpy as jnp
from jax.experimental import pallas as pl


def kernel(x, edge_index, batch, W1, as1, ad1, b1, W2, as2, ad2, b2, W3, as3, ad3, b3, W4, as4, ad4, b4, W5, as5, ad5, b5, W6, as6, ad6, b6, Wf, bf):
    raise NotImplementedError("write your pallas kernel here")


# speedup vs baseline: 18.8274x; 18.8274x over previous
import functools

import jax
import jax.numpy as jnp
from jax import lax
from jax.experimental import pallas as pl
from jax.experimental.pallas import tpu as pltpu
from jax.experimental.pallas import tpu_sc as plsc

N = 41472
E = 663552
B = 512
IN = 3
LAT = 32

ROWS = 512
GRID = N // ROWS
NSC = 2
NTILE = 16
SLAB = N // NTILE
QB = SLAB // 8
CH = 128
FG = 16


def _lrelu(v):
    return jnp.where(v > 0, v, 0.2 * v)


def _prep_body(ng, x_ref, w_ref, as_ref, ad_ref, hp_ref, s_ref, d_ref,
               ss_ref):
    i = pl.program_id(0)
    xb = x_ref[...]
    hp = jnp.dot(xb, w_ref[...], preferred_element_type=jnp.float32)
    s = jnp.sum(hp * as_ref[...][None, :], axis=1)
    d = jnp.sum(hp * ad_ref[...][None, :], axis=1)
    for j in range(ng):
        hp_ref[j] = hp[:, j * FG:(j + 1) * FG]
    s_ref[...] = s
    d_ref[...] = d
    bm = jnp.max(s)
    prev = jnp.where(i == 0, -jnp.inf, ss_ref[0])
    ss_ref[...] = jnp.full((128,), jnp.maximum(bm, prev), jnp.float32)


def _prep(x, W, a_s, a_d, ng):
    di = x.shape[1]
    D = ng * FG
    return pl.pallas_call(
        functools.partial(_prep_body, ng),
        grid=(GRID,),
        in_specs=[
            pl.BlockSpec((ROWS, di), lambda i: (i, 0)),
            pl.BlockSpec((di, D), lambda i: (0, 0)),
            pl.BlockSpec((D,), lambda i: (0,)),
            pl.BlockSpec((D,), lambda i: (0,)),
        ],
        out_specs=[
            pl.BlockSpec((ng, ROWS, FG), lambda i: (0, i, 0)),
            pl.BlockSpec((ROWS,), lambda i: (i,)),
            pl.BlockSpec((ROWS,), lambda i: (i,)),
            pl.BlockSpec((128,), lambda i: (0,)),
        ],
        out_shape=[
            jax.ShapeDtypeStruct((ng, N, FG), jnp.float32),
            jax.ShapeDtypeStruct((N,), jnp.float32),
            jax.ShapeDtypeStruct((N,), jnp.float32),
            jax.ShapeDtypeStruct((128,), jnp.float32),
        ],
    )(x, W, a_s, a_d)


def _edge_body(g0, esplit, hp_ref, s_ref, d_ref, ss_ref, src_ref, dst_ref,
               zf_ref, zd_ref, num_ref, den_ref, accf, accd, sh_hp, sh_s,
               sh_d, srcb0, srcb1, dstb0, dstb1, sg0, sg1, dg0, dg1, wb0,
               wb1, rows0, rows1, sbuf, zfb, zdb, semg0, semg1, sem):
    c = lax.axis_index("c")
    t_id = lax.axis_index("s")
    r0 = t_id * SLAB
    pltpu.sync_copy(zf_ref, zfb)
    pltpu.sync_copy(zd_ref, zdb)
    for h in range(8):
        pltpu.sync_copy(zfb, accf.at[pl.ds(r0 + h * QB, QB)])
    pltpu.sync_copy(zdb, accd.at[pl.ds(r0, SLAB)])
    g = g0 if esplit else g0 + c
    for h in range(8):
        pltpu.sync_copy(hp_ref.at[g, pl.ds(r0 + h * QB, QB)], zfb)
        pltpu.sync_copy(zfb, sh_hp.at[pl.ds(r0 + h * QB, QB)])
    pltpu.sync_copy(s_ref.at[pl.ds(r0, SLAB)], zdb)
    pltpu.sync_copy(zdb, sh_s.at[pl.ds(r0, SLAB)])
    pltpu.sync_copy(d_ref.at[pl.ds(r0, SLAB)], zdb)
    pltpu.sync_copy(zdb, sh_d.at[pl.ds(r0, SLAB)])
    pltpu.sync_copy(ss_ref.at[pl.ds(0, 16)], sbuf)
    plsc.subcore_barrier()
    if esplit:
        ept = E // (NSC * NTILE)
        base = (c * NTILE + t_id) * ept
    else:
        ept = E // NTILE
        base = t_id * ept
    nch = ept // CH
    srcb_ = (srcb0, srcb1)
    dstb_ = (dstb0, dstb1)
    sg_ = (sg0, sg1)
    dg_ = (dg0, dg1)
    wb_ = (wb0, wb1)
    rows_ = (rows0, rows1)
    semg_ = (semg0, semg1)

    def fetch(t, par):
        e0 = pl.multiple_of(base + t * CH, 128)
        pltpu.sync_copy(src_ref.at[pl.ds(e0, CH)], srcb_[par])
        pltpu.sync_copy(dst_ref.at[pl.ds(e0, CH)], dstb_[par])
        pltpu.async_copy(sh_s.at[srcb_[par]], sg_[par], semg_[par])
        pltpu.async_copy(sh_d.at[dstb_[par]], dg_[par], semg_[par])
        pltpu.async_copy(sh_hp.at[srcb_[par]], rows_[par], semg_[par])

    def process(par):
        pltpu.make_async_copy(sh_s.at[srcb_[par]], sg_[par],
                              semg_[par]).wait()
        pltpu.make_async_copy(sh_d.at[dstb_[par]], dg_[par],
                              semg_[par]).wait()
        pltpu.make_async_copy(sh_hp.at[srcb_[par]], rows_[par],
                              semg_[par]).wait()
        sg, dg, wb, rows = sg_[par], dg_[par], wb_[par], rows_[par]
        Sv = sbuf[...]
        for q in range(CH // 16):
            sv = sg[pl.ds(q * 16, 16)]
            dv = dg[pl.ds(q * 16, 16)]
            a = sv + dv
            a = jnp.where(a > 0, a, 0.2 * a)
            mm = Sv + dv
            mm = jnp.where(mm > 0, mm, 0.2 * mm)
            wb[pl.ds(q * 16, 16)] = jnp.exp(a - mm)
        for e in range(CH):
            wev = plsc.load_gather(wb, [jnp.full((16,), e, jnp.int32)])
            rows[e, :] = rows[e, :] * wev
        pltpu.async_copy(rows, accf.at[dstb_[par]], sem, add=True)
        pltpu.async_copy(wb, accd.at[dstb_[par]], sem, add=True)
        pltpu.make_async_copy(rows, accf.at[dstb_[par]], sem).wait()
        pltpu.make_async_copy(wb, accd.at[dstb_[par]], sem).wait()

    fetch(0, 0)

    def pair(i, carry):
        fetch(2 * i + 1, 1)
        process(0)

        @pl.when(i < nch // 2 - 1)
        def _():
            fetch(2 * i + 2, 0)

        process(1)
        return carry

    lax.fori_loop(0, nch // 2, pair, 0)
    plsc.subcore_barrier()
    for h in range(8):
        pltpu.sync_copy(accf.at[pl.ds(r0 + h * QB, QB)], zfb)
        pltpu.sync_copy(zfb, num_ref.at[c, pl.ds(r0 + h * QB, QB)])
    pltpu.sync_copy(accd.at[pl.ds(r0, SLAB)], zdb)
    pltpu.sync_copy(zdb, den_ref.at[pl.ds(c * N + r0, SLAB)])


def _edge_pass(hp_t, s, d, ss, src, dst, g0, esplit):
    mesh = plsc.VectorSubcoreMesh(core_axis_name="c", subcore_axis_name="s")
    zf = jnp.zeros((QB, FG), jnp.float32)
    zd = jnp.zeros((SLAB,), jnp.float32)
    k = pl.kernel(
        functools.partial(_edge_body, g0, esplit),
        out_type=[
            jax.ShapeDtypeStruct((NSC, N, FG), jnp.float32),
            jax.ShapeDtypeStruct((NSC * N,), jnp.float32),
        ],
        mesh=mesh,
        compiler_params=pltpu.CompilerParams(needs_layout_passes=False,
                                             use_tc_tiling_on_sc=False),
        scratch_types=[
            pltpu.VMEM_SHARED((N, FG), jnp.float32),
            pltpu.VMEM_SHARED((N,), jnp.float32),
            pltpu.VMEM_SHARED((N, FG), jnp.float32),
            pltpu.VMEM_SHARED((N,), jnp.float32),
            pltpu.VMEM_SHARED((N,), jnp.float32),
            pltpu.VMEM((CH,), jnp.int32),
            pltpu.VMEM((CH,), jnp.int32),
            pltpu.VMEM((CH,), jnp.int32),
            pltpu.VMEM((CH,), jnp.int32),
            pltpu.VMEM((CH,), jnp.float32),
            pltpu.VMEM((CH,), jnp.float32),
            pltpu.VMEM((CH,), jnp.float32),
            pltpu.VMEM((CH,), jnp.float32),
            pltpu.VMEM((CH,), jnp.float32),
            pltpu.VMEM((CH,), jnp.float32),
            pltpu.VMEM((CH, FG), jnp.float32),
            pltpu.VMEM((CH, FG), jnp.float32),
            pltpu.VMEM((16,), jnp.float32),
            pltpu.VMEM((QB, FG), jnp.float32),
            pltpu.VMEM((SLAB,), jnp.float32),
            pltpu.SemaphoreType.DMA,
            pltpu.SemaphoreType.DMA,
            pltpu.SemaphoreType.DMA,
        ],
    )
    num, den = k(hp_t, s, d, ss, src, dst, zf, zd)
    return num, den.reshape(NSC, N)


def _fin_body(ng, esplit, act, num_ref, den_ref, hp_ref, s_ref, d_ref,
              ss_ref, b_ref, out_ref):
    S = ss_ref[0]
    s = s_ref[...]
    d = d_ref[...]
    m = _lrelu(S + d)
    wself = jnp.exp(_lrelu(s + d) - m)
    hp = jnp.concatenate([hp_ref[j] for j in range(ng)], axis=1)
    if esplit:
        num = num_ref[0] + num_ref[1]
        den = den_ref[0] + den_ref[1]
    else:
        num = jnp.concatenate([num_ref[j] for j in range(ng)], axis=1)
        den = den_ref[0]
    num = num + wself[:, None] * hp
    den = den + wself
    out = num / den[:, None] + b_ref[...][None, :]
    if act == "relu":
        out = jnp.maximum(out, 0.0)
    elif act == "final":
        ci = lax.broadcasted_iota(jnp.int32, out.shape, 1)
        out = jnp.where(ci < 2, jnp.tanh(out), jnp.maximum(out, 0.0))
    out_ref[...] = out


def _finalize(num, den, hp_t, s, d, ss, b, ng, esplit, act):
    D = ng * FG
    nk = num.shape[0]
    return pl.pallas_call(
        functools.partial(_fin_body, ng, esplit, act),
        grid=(GRID,),
        in_specs=[
            pl.BlockSpec((nk, ROWS, FG), lambda i: (0, i, 0)),
            pl.BlockSpec((NSC, ROWS), lambda i: (0, i)),
            pl.BlockSpec((ng, ROWS, FG), lambda i: (0, i, 0)),
            pl.BlockSpec((ROWS,), lambda i: (i,)),
            pl.BlockSpec((ROWS,), lambda i: (i,)),
            pl.BlockSpec((128,), lambda i: (0,)),
            pl.BlockSpec((D,), lambda i: (0,)),
        ],
        out_specs=pl.BlockSpec((ROWS, D), lambda i: (i, 0)),
        out_shape=jax.ShapeDtypeStruct((N, D), jnp.float32),
    )(num, den, hp_t, s, d, ss, b)


def _gat(h, W, a_s, a_d, b, src, dst, ng, act):
    D = ng * FG
    do = W.shape[1]
    if do < D:
        W = jnp.pad(W, ((0, 0), (0, D - do)))
        a_s = jnp.pad(a_s, (0, D - do))
        a_d = jnp.pad(a_d, (0, D - do))
        b = jnp.pad(b, (0, D - do))
    hp_t, s, d, ss = _prep(h, W, a_s, a_d, ng)
    esplit = ng == 1
    nums = []
    den = None
    for p in range(max(1, ng // 2)):
        num_p, den_p = _edge_pass(hp_t, s, d, ss, src, dst, 2 * p, esplit)
        nums.append(num_p)
        if den is None:
            den = den_p
    num = jnp.concatenate(nums, axis=0) if len(nums) > 1 else nums[0]
    return _finalize(num, den, hp_t, s, d, ss, b, ng, esplit, act)


PCH = 48
PT = N // (NSC * NTILE)
PSLAB = B // NTILE


def _pool_body(x_ref, batch_ref, zf_ref, zd_ref, acc_ref, cnt_ref,
               acc, cnt, xbuf, bbuf, ones, zfb, zdb, sem):
    c = lax.axis_index("c")
    t_id = lax.axis_index("s")
    r0 = t_id * PSLAB
    pltpu.sync_copy(zf_ref, zfb)
    pltpu.sync_copy(zd_ref, zdb)
    pltpu.sync_copy(zfb, acc.at[pl.ds(r0, PSLAB)])
    pltpu.sync_copy(zdb, cnt.at[pl.ds(r0, PSLAB)])
    for q in range(PCH // 16):
        ones[pl.ds(q * 16, 16)] = jnp.full((16,), 1.0, jnp.float32)
    plsc.subcore_barrier()
    base = c * (N // NSC) + t_id * PT

    def chunk(t, carry):
        n0 = pl.multiple_of(base + t * PCH, 8)
        pltpu.sync_copy(x_ref.at[pl.ds(n0, PCH)], xbuf)
        pltpu.sync_copy(batch_ref.at[pl.ds(n0, PCH)], bbuf)
        pltpu.sync_copy(xbuf, acc.at[bbuf], add=True)
        pltpu.sync_copy(ones, cnt.at[bbuf], add=True)
        return carry

    lax.fori_loop(0, PT // PCH, chunk, 0)
    plsc.subcore_barrier()
    pltpu.sync_copy(acc.at[pl.ds(r0, PSLAB)], zfb)
    pltpu.sync_copy(zfb, acc_ref.at[c, pl.ds(r0, PSLAB)])
    pltpu.sync_copy(cnt.at[pl.ds(r0, PSLAB)], zdb)
    pltpu.sync_copy(zdb, cnt_ref.at[pl.ds(c * B + r0, PSLAB)])


def _pool(x3, batch):
    mesh = plsc.VectorSubcoreMesh(core_axis_name="c", subcore_axis_name="s")
    zf = jnp.zeros((PSLAB, LAT), jnp.float32)
    zd = jnp.zeros((PSLAB,), jnp.float32)
    k = pl.kernel(
        _pool_body,
        out_type=[
            jax.ShapeDtypeStruct((NSC, B, LAT), jnp.float32),
            jax.ShapeDtypeStruct((NSC * B,), jnp.float32),
        ],
        mesh=mesh,
        compiler_params=pltpu.CompilerParams(needs_layout_passes=False,
                                             use_tc_tiling_on_sc=False),
        scratch_types=[
            pltpu.VMEM_SHARED((B, LAT), jnp.float32),
            pltpu.VMEM_SHARED((B,), jnp.float32),
            pltpu.VMEM((PCH, LAT), jnp.float32),
            pltpu.VMEM((PCH,), jnp.int32),
            pltpu.VMEM((PCH,), jnp.float32),
            pltpu.VMEM((PSLAB, LAT), jnp.float32),
            pltpu.VMEM((PSLAB,), jnp.float32),
            pltpu.SemaphoreType.DMA,
        ],
    )
    pacc, pcnt = k(x3, batch, zf, zd)
    return pacc, pcnt.reshape(NSC, B)


def _expand_body(acc_ref, cnt_ref, wf_ref, bf_ref, xg_ref, xe_ref):
    accs = acc_ref[0] + acc_ref[1]
    cnt = jnp.maximum(cnt_ref[0] + cnt_ref[1], 1.0)
    xg = accs / cnt[:, None]
    xg_ref[...] = xg
    xe_ref[...] = (jnp.dot(xg, wf_ref[...], preferred_element_type=jnp.float32)
                   + bf_ref[...][None, :])


def _expand(pacc, pcnt, Wf, bf):
    F = Wf.shape[1]
    return pl.pallas_call(
        _expand_body,
        out_shape=[
            jax.ShapeDtypeStruct((B, LAT), jnp.float32),
            jax.ShapeDtypeStruct((B, F), jnp.float32),
        ],
    )(pacc, pcnt, Wf, bf)


def kernel(x, edge_index, batch, W1, as1, ad1, b1, W2, as2, ad2, b2, W3, as3,
           ad3, b3, W4, as4, ad4, b4, W5, as5, ad5, b5, W6, as6, ad6, b6,
           Wf, bf):
    src = edge_index[0]
    dst = edge_index[1]
    x1 = _gat(x, W1, as1, ad1, b1, src, dst, 2, "relu")
    x2 = _gat(x1, W2, as2, ad2, b2, src, dst, 4, "relu")
    x3 = _gat(x2, W3, as3, ad3, b3, src, dst, 2, "relu")
    pacc, pcnt = _pool(x3, batch)
    xg, xe = _expand(pacc, pcnt, Wf, bf)
    xe_nodes = xe.reshape(N, IN)
    xr = _gat(xe_nodes, W4, as4, ad4, b4, src, dst, 4, "relu")
    xr = _gat(xr, W5, as5, ad5, b5, src, dst, 2, "relu")
    xr = _gat(xr, W6, as6, ad6, b6, src, dst, 1, "final")
    return (xr[:, :IN], xg)
